# baseline (speedup 1.0000x reference)
import functools

import jax
import jax.numpy as jnp
from jax import lax
from jax.experimental import pallas as pl
from jax.experimental.pallas import tpu as pltpu
from jax.experimental.pallas import tpu_sc as plsc

T = 2048
C = 768
E = 16
FF = 3072
TM = 256
NT = T // TM + (E - 1)
NPAD = NT * TM
NW = 32
TPW = T // NW
LANES = 16


def _gate_body(x_ref, wg_ref, bg_ref, pos_ref, meta_ref, f_ref, p_ref):
    x = x_ref[...]
    logits = jnp.dot(x, wg_ref[...], preferred_element_type=jnp.float32)
    s = jax.nn.sigmoid(logits + bg_ref[...])
    m = jnp.max(s, axis=1, keepdims=True)
    lane_e = lax.broadcasted_iota(jnp.int32, (1, E), 1)
    cand = jnp.where(s >= m, lane_e, E)
    e_t = jnp.min(cand, axis=1, keepdims=True)
    onehot = (lane_e == e_t).astype(jnp.float32)

    denom = jnp.sum(s, axis=1, keepdims=True)
    s_sel = m / denom
    counts = jnp.sum(onehot, axis=0, keepdims=True)
    f_ref[...] = jnp.float32(T) - counts
    sel_per_e = jnp.sum(onehot * s_sel, axis=0, keepdims=True)
    p_ref[...] = jnp.sum(s_sel) - sel_per_e

    tiles = jnp.floor((counts + jnp.float32(TM - 1)) * jnp.float32(1.0 / TM))
    r16 = lax.broadcasted_iota(jnp.int32, (E, E), 0)
    c16 = lax.broadcasted_iota(jnp.int32, (E, E), 1)
    excl = (r16 < c16).astype(jnp.float32)
    ts_row = jnp.dot(tiles, excl, preferred_element_type=jnp.float32)
    nu = jnp.sum(tiles, axis=1, keepdims=True)

    BL = 256
    r_b = lax.broadcasted_iota(jnp.int32, (BL, BL), 0)
    c_b = lax.broadcasted_iota(jnp.int32, (BL, BL), 1)
    tri = (r_b >= c_b).astype(jnp.float32)
    ranks = []
    off = jnp.zeros((1, E), jnp.float32)
    for i in range(T // BL):
        blk = onehot[i * BL:(i + 1) * BL]
        cum = jnp.dot(tri, blk, preferred_element_type=jnp.float32) + off
        off = off + jnp.sum(blk, axis=0, keepdims=True)
        ranks.append(jnp.sum(blk * cum, axis=1, keepdims=True) - 1.0)
    rank = jnp.concatenate(ranks, axis=0)

    ts_t = jnp.sum(onehot * ts_row, axis=1, keepdims=True)
    pos_ref[...] = (jnp.float32(TM) * ts_t + rank).astype(jnp.int32)

    lane = lax.broadcasted_iota(jnp.int32, (1, 128), 1)
    nu_i = nu.astype(jnp.int32)
    g1 = jnp.minimum(lane, nu_i - 1)
    acc = jnp.zeros((1, 128), jnp.int32)
    for e in range(E):
        ts_e = ts_row[:, e:e + 1].astype(jnp.int32)
        acc = acc + (g1 >= ts_e).astype(jnp.int32)
    eot = acc - 1
    tidx = jnp.minimum(lane - 32, nu_i - 1)
    meta_ref[...] = jnp.where(lane < 32, eot,
                              jnp.where(lane < 64, tidx, nu_i))


def _run_gate(xf, Wg, bg2, interpret=False):
    return pl.pallas_call(
        _gate_body,
        out_shape=(
            jax.ShapeDtypeStruct((T, 1), jnp.int32),
            jax.ShapeDtypeStruct((1, 128), jnp.int32),
            jax.ShapeDtypeStruct((1, E), jnp.float32),
            jax.ShapeDtypeStruct((1, E), jnp.float32),
        ),
        interpret=interpret,
    )(xf, Wg, bg2)


def _shared_body(x_ref, ws_ref, bs_ref, base_ref):
    x = x_ref[...]
    ws = ws_ref[...]
    bs = bs_ref[...]
    ws_eff = (ws[:, :C] + ws[:, C:]).astype(jnp.bfloat16)
    bs_eff = bs[:, :C] + bs[:, C:]
    xb = x.astype(jnp.bfloat16)
    base_ref[...] = (
        x + jnp.dot(xb, ws_eff, preferred_element_type=jnp.float32) + bs_eff)


def _run_shared(xf, Ws, bs2, interpret=False):
    return pl.pallas_call(
        _shared_body,
        out_shape=jax.ShapeDtypeStruct((T, C), jnp.float32),
        interpret=interpret,
    )(xf, Ws, bs2)


def _ffn_body(meta_ref, x_ref, w1_ref, b1_ref, w2_ref, b2_ref, y_ref):
    g = pl.program_id(0)

    @pl.when(g < meta_ref[64])
    def _():
        xb = x_ref[...].astype(jnp.bfloat16)
        w1 = w1_ref[0].astype(jnp.bfloat16)
        h = jnp.dot(xb, w1, preferred_element_type=jnp.float32)
        h = jax.nn.gelu(h + b1_ref[0]).astype(jnp.bfloat16)
        w2 = w2_ref[0].astype(jnp.bfloat16)
        y = jnp.dot(h, w2, preferred_element_type=jnp.float32)
        y_ref[...] = y + b2_ref[0]


def _run_ffn(meta, xpad, W1, b1r, W2, b2r, interpret=False):
    grid_spec = pltpu.PrefetchScalarGridSpec(
        num_scalar_prefetch=1,
        grid=(NT,),
        in_specs=[
            pl.BlockSpec((TM, C), lambda g, m: (m[32 + g], 0)),
            pl.BlockSpec((1, C, FF), lambda g, m: (m[g], 0, 0)),
            pl.BlockSpec((1, 1, FF), lambda g, m: (m[g], 0, 0)),
            pl.BlockSpec((1, FF, C), lambda g, m: (m[g], 0, 0)),
            pl.BlockSpec((1, 1, C), lambda g, m: (m[g], 0, 0)),
        ],
        out_specs=pl.BlockSpec((TM, C), lambda g, m: (m[32 + g], 0)),
    )
    return pl.pallas_call(
        _ffn_body,
        grid_spec=grid_spec,
        out_shape=jax.ShapeDtypeStruct((NPAD, C), jnp.float32),
        compiler_params=pltpu.CompilerParams(
            dimension_semantics=("arbitrary",)),
        interpret=interpret,
    )(meta, xpad, W1, b1r, W2, b2r)


@functools.cache
def _sc_kernels():
    mesh = plsc.VectorSubcoreMesh(core_axis_name="c", subcore_axis_name="s")

    @functools.partial(
        pl.kernel,
        out_type=jax.ShapeDtypeStruct((NPAD, C), jnp.float32),
        mesh=mesh,
        scratch_types=[pltpu.VMEM((TPW,), jnp.int32),
                       pltpu.VMEM((TPW, C), jnp.float32),
                       pltpu.SemaphoreType.DMA,
                       pltpu.SemaphoreType.DMA],
    )
    def _sc_scatter(x_hbm, pos_hbm, xpad_hbm, idx_v, buf_v, sem1, sem2):
        wid = lax.axis_index("s") * 2 + lax.axis_index("c")
        start = wid * TPW
        cpx = pltpu.async_copy(x_hbm.at[pl.ds(start, TPW)], buf_v, sem1)
        pltpu.sync_copy(pos_hbm.at[pl.ds(start, TPW)], idx_v)
        cpx.wait()
        pltpu.async_copy(buf_v, xpad_hbm.at[idx_v], sem2).wait()

    @functools.partial(
        pl.kernel,
        out_type=jax.ShapeDtypeStruct((T, C), jnp.float32),
        mesh=mesh,
        scratch_types=[pltpu.VMEM((TPW,), jnp.int32),
                       pltpu.VMEM((TPW, C), jnp.float32),
                       pltpu.VMEM((TPW, C), jnp.float32),
                       pltpu.SemaphoreType.DMA,
                       pltpu.SemaphoreType.DMA],
    )
    def _sc_gather(ypad_hbm, base_hbm, pos_hbm, res_hbm,
                   idx_v, bufy_v, bufb_v, sem1, sem2):
        wid = lax.axis_index("s") * 2 + lax.axis_index("c")
        start = wid * TPW
        cpb = pltpu.async_copy(base_hbm.at[pl.ds(start, TPW)], bufb_v, sem1)
        pltpu.sync_copy(pos_hbm.at[pl.ds(start, TPW)], idx_v)
        cpy = pltpu.async_copy(ypad_hbm.at[idx_v], bufy_v, sem2)
        cpb.wait()
        cpy.wait()

        def row_add(i, _):
            for j in range(C // LANES):
                sl = pl.ds(j * LANES, LANES)
                bufy_v[i, sl] = bufy_v[i, sl] + bufb_v[i, sl]
            return _

        lax.fori_loop(0, TPW, row_add, 0)
        pltpu.sync_copy(bufy_v, res_hbm.at[pl.ds(start, TPW)])

    return _sc_scatter, _sc_gather


def kernel(x, Ws, bs, Wg, bg, W1, b1, W2, b2):
    xf = x.reshape(T, C)
    pos, meta, f, p = _run_gate(xf, Wg, bg.reshape(1, -1))
    base = _run_shared(xf, Ws, bs.reshape(1, -1))
    pos1 = pos.reshape(T)
    _sc_scatter, _sc_gather = _sc_kernels()
    xpad = _sc_scatter(xf, pos1)
    ypad = _run_ffn(meta.reshape(128), xpad, W1,
                    b1.reshape(E, 1, FF), W2, b2.reshape(E, 1, C))
    res = _sc_gather(ypad, base, pos1)
    return res.reshape(1, T, C), (f, p)


# speedup vs baseline: 1.0563x; 1.0563x over previous
import functools

import jax
import jax.numpy as jnp
from jax import lax
from jax.experimental import pallas as pl
from jax.experimental.pallas import tpu as pltpu
from jax.experimental.pallas import tpu_sc as plsc

T = 2048
C = 768
E = 16
FF = 3072
TM = 256
NT = T // TM + (E - 1)
NPAD = NT * TM
NW = 32
TPW = T // NW


def _gate_body(x_ref, wg_ref, bg_ref, ws_ref, bs_ref,
               pos_ref, meta_ref, f_ref, p_ref, wse_ref, bse_ref):
    x = x_ref[...]
    logits = jnp.dot(x, wg_ref[...], preferred_element_type=jnp.float32)
    s = jax.nn.sigmoid(logits + bg_ref[...])
    m = jnp.max(s, axis=1, keepdims=True)
    lane_e = lax.broadcasted_iota(jnp.int32, (1, E), 1)
    cand = jnp.where(s >= m, lane_e, E)
    e_t = jnp.min(cand, axis=1, keepdims=True)
    onehot = (lane_e == e_t).astype(jnp.float32)

    denom = jnp.sum(s, axis=1, keepdims=True)
    s_sel = m / denom
    counts = jnp.sum(onehot, axis=0, keepdims=True)
    f_ref[...] = jnp.float32(T) - counts
    sel_per_e = jnp.sum(onehot * s_sel, axis=0, keepdims=True)
    p_ref[...] = jnp.sum(s_sel) - sel_per_e

    ws = ws_ref[...]
    bs = bs_ref[...]
    wse_ref[...] = (ws[:, :C] + ws[:, C:]).astype(jnp.bfloat16)
    bse_ref[...] = bs[:, :C] + bs[:, C:]

    tiles = jnp.floor((counts + jnp.float32(TM - 1)) * jnp.float32(1.0 / TM))
    r16 = lax.broadcasted_iota(jnp.int32, (E, E), 0)
    c16 = lax.broadcasted_iota(jnp.int32, (E, E), 1)
    excl = (r16 < c16).astype(jnp.float32)
    ts_row = jnp.dot(tiles, excl, preferred_element_type=jnp.float32)
    nu = jnp.sum(tiles, axis=1, keepdims=True)

    BL = 256
    r_b = lax.broadcasted_iota(jnp.int32, (BL, BL), 0)
    c_b = lax.broadcasted_iota(jnp.int32, (BL, BL), 1)
    tri = (r_b >= c_b).astype(jnp.float32)
    ranks = []
    off = jnp.zeros((1, E), jnp.float32)
    for i in range(T // BL):
        blk = onehot[i * BL:(i + 1) * BL]
        cum = jnp.dot(tri, blk, preferred_element_type=jnp.float32) + off
        off = off + jnp.sum(blk, axis=0, keepdims=True)
        ranks.append(jnp.sum(blk * cum, axis=1, keepdims=True) - 1.0)
    rank = jnp.concatenate(ranks, axis=0)

    ts_t = jnp.sum(onehot * ts_row, axis=1, keepdims=True)
    pos_ref[...] = (jnp.float32(TM) * ts_t + rank).astype(jnp.int32)

    lane = lax.broadcasted_iota(jnp.int32, (1, 128), 1)
    nu_i = nu.astype(jnp.int32)
    g1 = jnp.minimum(lane, nu_i - 1)
    acc = jnp.zeros((1, 128), jnp.int32)
    for e in range(E):
        ts_e = ts_row[:, e:e + 1].astype(jnp.int32)
        acc = acc + (g1 >= ts_e).astype(jnp.int32)
    eot = acc - 1
    tidx = jnp.minimum(lane - 32, nu_i - 1)
    meta_ref[...] = jnp.where(lane < 32, eot,
                              jnp.where(lane < 64, tidx, nu_i))


def _run_gate(xf, Wg, bg2, Ws, bs2, interpret=False):
    return pl.pallas_call(
        _gate_body,
        out_shape=(
            jax.ShapeDtypeStruct((T, 1), jnp.int32),
            jax.ShapeDtypeStruct((1, 128), jnp.int32),
            jax.ShapeDtypeStruct((1, E), jnp.float32),
            jax.ShapeDtypeStruct((1, E), jnp.float32),
            jax.ShapeDtypeStruct((C, C), jnp.bfloat16),
            jax.ShapeDtypeStruct((1, C), jnp.float32),
        ),
        interpret=interpret,
    )(xf, Wg, bg2, Ws, bs2)


def _ffn_body(meta_ref, x_ref, w1_ref, b1_ref, w2_ref, b2_ref,
              wse_ref, bse_ref, y_ref):
    g = pl.program_id(0)

    @pl.when(g < meta_ref[64])
    def _():
        xf = x_ref[...]
        xb = xf.astype(jnp.bfloat16)
        w1 = w1_ref[0].astype(jnp.bfloat16)
        h = jnp.dot(xb, w1, preferred_element_type=jnp.float32)
        h = jax.nn.gelu(h + b1_ref[0]).astype(jnp.bfloat16)
        w2 = w2_ref[0].astype(jnp.bfloat16)
        y = jnp.dot(h, w2, preferred_element_type=jnp.float32)
        shared = jnp.dot(xb, wse_ref[...], preferred_element_type=jnp.float32)
        y_ref[...] = xf + shared + y + b2_ref[0] + bse_ref[...]


def _run_ffn(meta, xpad, W1, b1r, W2, b2r, wse, bse, interpret=False):
    grid_spec = pltpu.PrefetchScalarGridSpec(
        num_scalar_prefetch=1,
        grid=(NT,),
        in_specs=[
            pl.BlockSpec((TM, C), lambda g, m: (m[32 + g], 0)),
            pl.BlockSpec((1, C, FF), lambda g, m: (m[g], 0, 0)),
            pl.BlockSpec((1, 1, FF), lambda g, m: (m[g], 0, 0)),
            pl.BlockSpec((1, FF, C), lambda g, m: (m[g], 0, 0)),
            pl.BlockSpec((1, 1, C), lambda g, m: (m[g], 0, 0)),
            pl.BlockSpec((C, C), lambda g, m: (0, 0)),
            pl.BlockSpec((1, C), lambda g, m: (0, 0)),
        ],
        out_specs=pl.BlockSpec((TM, C), lambda g, m: (m[32 + g], 0)),
    )
    return pl.pallas_call(
        _ffn_body,
        grid_spec=grid_spec,
        out_shape=jax.ShapeDtypeStruct((NPAD, C), jnp.float32),
        compiler_params=pltpu.CompilerParams(
            dimension_semantics=("arbitrary",)),
        interpret=interpret,
    )(meta, xpad, W1, b1r, W2, b2r, wse, bse)


@functools.cache
def _sc_kernels():
    mesh = plsc.VectorSubcoreMesh(core_axis_name="c", subcore_axis_name="s")

    @functools.partial(
        pl.kernel,
        out_type=jax.ShapeDtypeStruct((NPAD, C), jnp.float32),
        mesh=mesh,
        scratch_types=[pltpu.VMEM((TPW,), jnp.int32),
                       pltpu.VMEM((TPW, C), jnp.float32),
                       pltpu.SemaphoreType.DMA,
                       pltpu.SemaphoreType.DMA],
    )
    def _sc_scatter(x_hbm, pos_hbm, xpad_hbm, idx_v, buf_v, sem1, sem2):
        wid = lax.axis_index("s") * 2 + lax.axis_index("c")
        start = wid * TPW
        cpx = pltpu.async_copy(x_hbm.at[pl.ds(start, TPW)], buf_v, sem1)
        pltpu.sync_copy(pos_hbm.at[pl.ds(start, TPW)], idx_v)
        cpx.wait()
        pltpu.async_copy(buf_v, xpad_hbm.at[idx_v], sem2).wait()

    @functools.partial(
        pl.kernel,
        out_type=jax.ShapeDtypeStruct((T, C), jnp.float32),
        mesh=mesh,
        scratch_types=[pltpu.VMEM((TPW,), jnp.int32),
                       pltpu.VMEM((TPW, C), jnp.float32),
                       pltpu.SemaphoreType.DMA],
    )
    def _sc_gather(ypad_hbm, pos_hbm, res_hbm, idx_v, buf_v, sem):
        wid = lax.axis_index("s") * 2 + lax.axis_index("c")
        start = wid * TPW
        pltpu.sync_copy(pos_hbm.at[pl.ds(start, TPW)], idx_v)
        pltpu.async_copy(ypad_hbm.at[idx_v], buf_v, sem).wait()
        pltpu.sync_copy(buf_v, res_hbm.at[pl.ds(start, TPW)])

    return _sc_scatter, _sc_gather


def kernel(x, Ws, bs, Wg, bg, W1, b1, W2, b2):
    xf = x.reshape(T, C)
    pos, meta, f, p, wse, bse = _run_gate(
        xf, Wg, bg.reshape(1, -1), Ws, bs.reshape(1, -1))
    pos1 = pos.reshape(T)
    _sc_scatter, _sc_gather = _sc_kernels()
    xpad = _sc_scatter(xf, pos1)
    ypad = _run_ffn(meta.reshape(128), xpad, W1,
                    b1.reshape(E, 1, FF), W2, b2.reshape(E, 1, C), wse, bse)
    res = _sc_gather(ypad, pos1)
    return res.reshape(1, T, C), (f, p)


# speedup vs baseline: 1.0578x; 1.0014x over previous
import functools

import jax
import jax.numpy as jnp
from jax import lax
from jax.experimental import pallas as pl
from jax.experimental.pallas import tpu as pltpu
from jax.experimental.pallas import tpu_sc as plsc

T = 2048
C = 768
E = 16
FF = 3072
TM = 256
NT = T // TM + (E - 1)
NPAD = NT * TM
NW = 32
TPW = T // NW


def _gate_body(x_ref, wg_ref, bg_ref, ws_ref, bs_ref,
               pos_ref, meta_ref, f_ref, p_ref, wse_ref, bse_ref):
    x = x_ref[0]
    logits = jnp.dot(x, wg_ref[...], preferred_element_type=jnp.float32)
    s = jax.nn.sigmoid(logits + bg_ref[...])
    m = jnp.max(s, axis=1, keepdims=True)
    lane_e = lax.broadcasted_iota(jnp.int32, (1, E), 1)
    cand = jnp.where(s >= m, lane_e, E)
    e_t = jnp.min(cand, axis=1, keepdims=True)
    onehot = (lane_e == e_t).astype(jnp.float32)

    denom = jnp.sum(s, axis=1, keepdims=True)
    s_sel = m / denom
    counts = jnp.sum(onehot, axis=0, keepdims=True)
    f_ref[...] = jnp.float32(T) - counts
    sel_per_e = jnp.sum(onehot * s_sel, axis=0, keepdims=True)
    p_ref[...] = jnp.sum(s_sel) - sel_per_e

    ws = ws_ref[...]
    bs = bs_ref[...]
    wse_ref[...] = (ws[:, :C] + ws[:, C:]).astype(jnp.bfloat16)
    bse_ref[...] = bs[:, :C] + bs[:, C:]

    tiles = jnp.floor((counts + jnp.float32(TM - 1)) * jnp.float32(1.0 / TM))
    r16 = lax.broadcasted_iota(jnp.int32, (E, E), 0)
    c16 = lax.broadcasted_iota(jnp.int32, (E, E), 1)
    excl = (r16 < c16).astype(jnp.float32)
    ts_row = jnp.dot(tiles, excl, preferred_element_type=jnp.float32)
    nu = jnp.sum(tiles, axis=1, keepdims=True)

    BL = 256
    r_b = lax.broadcasted_iota(jnp.int32, (BL, BL), 0)
    c_b = lax.broadcasted_iota(jnp.int32, (BL, BL), 1)
    tri = (r_b >= c_b).astype(jnp.float32)
    ranks = []
    off = jnp.zeros((1, E), jnp.float32)
    for i in range(T // BL):
        blk = onehot[i * BL:(i + 1) * BL]
        cum = jnp.dot(tri, blk, preferred_element_type=jnp.float32) + off
        off = off + jnp.sum(blk, axis=0, keepdims=True)
        ranks.append(jnp.sum(blk * cum, axis=1, keepdims=True) - 1.0)
    rank = jnp.concatenate(ranks, axis=0)

    ts_t = jnp.sum(onehot * ts_row, axis=1, keepdims=True)
    pos_ref[...] = (jnp.float32(TM) * ts_t + rank).astype(jnp.int32)

    lane = lax.broadcasted_iota(jnp.int32, (1, 128), 1)
    nu_i = nu.astype(jnp.int32)
    g1 = jnp.minimum(lane, nu_i - 1)
    acc = jnp.zeros((1, 128), jnp.int32)
    for e in range(E):
        ts_e = ts_row[:, e:e + 1].astype(jnp.int32)
        acc = acc + (g1 >= ts_e).astype(jnp.int32)
    eot = acc - 1
    tidx = jnp.minimum(lane - 32, nu_i - 1)
    meta_ref[...] = jnp.where(lane < 32, eot,
                              jnp.where(lane < 64, tidx, nu_i))


def _run_gate(x3, Wg, bg2, Ws, bs2, interpret=False):
    return pl.pallas_call(
        _gate_body,
        out_shape=(
            jax.ShapeDtypeStruct((T, 1), jnp.int32),
            jax.ShapeDtypeStruct((1, 128), jnp.int32),
            jax.ShapeDtypeStruct((1, E), jnp.float32),
            jax.ShapeDtypeStruct((1, E), jnp.float32),
            jax.ShapeDtypeStruct((C, C), jnp.bfloat16),
            jax.ShapeDtypeStruct((1, C), jnp.float32),
        ),
        interpret=interpret,
    )(x3, Wg, bg2, Ws, bs2)


def _ffn_body(meta_ref, x_ref, w1_ref, b1_ref, w2_ref, b2_ref,
              wse_ref, bse_ref, y_ref):
    g = pl.program_id(0)

    @pl.when(g < meta_ref[64])
    def _():
        xf = x_ref[...]
        xb = xf.astype(jnp.bfloat16)
        w1 = w1_ref[0].astype(jnp.bfloat16)
        h = jnp.dot(xb, w1, preferred_element_type=jnp.float32)
        h = jax.nn.gelu(h + b1_ref[0]).astype(jnp.bfloat16)
        w2 = w2_ref[0].astype(jnp.bfloat16)
        y = jnp.dot(h, w2, preferred_element_type=jnp.float32)
        shared = jnp.dot(xb, wse_ref[...], preferred_element_type=jnp.float32)
        y_ref[...] = xf + shared + y + b2_ref[0] + bse_ref[...]


def _run_ffn(meta, xpad, W1, b1r, W2, b2r, wse, bse, interpret=False):
    grid_spec = pltpu.PrefetchScalarGridSpec(
        num_scalar_prefetch=1,
        grid=(NT,),
        in_specs=[
            pl.BlockSpec((TM, C), lambda g, m: (m[32 + g], 0)),
            pl.BlockSpec((1, C, FF), lambda g, m: (m[g], 0, 0)),
            pl.BlockSpec((1, 1, FF), lambda g, m: (m[g], 0, 0)),
            pl.BlockSpec((1, FF, C), lambda g, m: (m[g], 0, 0)),
            pl.BlockSpec((1, 1, C), lambda g, m: (m[g], 0, 0)),
            pl.BlockSpec((C, C), lambda g, m: (0, 0)),
            pl.BlockSpec((1, C), lambda g, m: (0, 0)),
        ],
        out_specs=pl.BlockSpec((TM, C), lambda g, m: (m[32 + g], 0)),
    )
    return pl.pallas_call(
        _ffn_body,
        grid_spec=grid_spec,
        out_shape=jax.ShapeDtypeStruct((NPAD, C), jnp.float32),
        compiler_params=pltpu.CompilerParams(
            dimension_semantics=("arbitrary",)),
        interpret=interpret,
    )(meta, xpad, W1, b1r, W2, b2r, wse, bse)


@functools.cache
def _sc_kernels():
    mesh = plsc.VectorSubcoreMesh(core_axis_name="c", subcore_axis_name="s")

    @functools.partial(
        pl.kernel,
        out_type=jax.ShapeDtypeStruct((NPAD, C), jnp.float32),
        mesh=mesh,
        scratch_types=[pltpu.VMEM((TPW,), jnp.int32),
                       pltpu.VMEM((TPW, C), jnp.float32),
                       pltpu.SemaphoreType.DMA,
                       pltpu.SemaphoreType.DMA],
    )
    def _sc_scatter(x_hbm, pos_hbm, xpad_hbm, idx_v, buf_v, sem1, sem2):
        wid = lax.axis_index("s") * 2 + lax.axis_index("c")
        start = wid * TPW
        cpx = pltpu.async_copy(x_hbm.at[0, pl.ds(start, TPW)], buf_v, sem1)
        pltpu.sync_copy(pos_hbm.at[pl.ds(start, TPW)], idx_v)
        cpx.wait()
        pltpu.async_copy(buf_v, xpad_hbm.at[idx_v], sem2).wait()

    @functools.partial(
        pl.kernel,
        out_type=jax.ShapeDtypeStruct((1, T, C), jnp.float32),
        mesh=mesh,
        scratch_types=[pltpu.VMEM((TPW,), jnp.int32),
                       pltpu.VMEM((TPW, C), jnp.float32),
                       pltpu.SemaphoreType.DMA],
    )
    def _sc_gather(ypad_hbm, pos_hbm, res_hbm, idx_v, buf_v, sem):
        wid = lax.axis_index("s") * 2 + lax.axis_index("c")
        start = wid * TPW
        pltpu.sync_copy(pos_hbm.at[pl.ds(start, TPW)], idx_v)
        pltpu.async_copy(ypad_hbm.at[idx_v], buf_v, sem).wait()
        pltpu.sync_copy(buf_v, res_hbm.at[0, pl.ds(start, TPW)])

    return _sc_scatter, _sc_gather


def kernel(x, Ws, bs, Wg, bg, W1, b1, W2, b2):
    pos, meta, f, p, wse, bse = _run_gate(
        x, Wg, bg.reshape(1, -1), Ws, bs.reshape(1, -1))
    pos1 = pos.reshape(T)
    _sc_scatter, _sc_gather = _sc_kernels()
    xpad = _sc_scatter(x, pos1)
    ypad = _run_ffn(meta.reshape(128), xpad, W1,
                    b1.reshape(E, 1, FF), W2, b2.reshape(E, 1, C), wse, bse)
    res = _sc_gather(ypad, pos1)
    return res, (f, p)


# speedup vs baseline: 1.0602x; 1.0023x over previous
import functools

import jax
import jax.numpy as jnp
from jax import lax
from jax.experimental import pallas as pl
from jax.experimental.pallas import tpu as pltpu
from jax.experimental.pallas import tpu_sc as plsc

T = 2048
C = 768
E = 16
FF = 3072
TM = 256
NT = T // TM + (E - 1)
NPAD = NT * TM
NW = 32
TPW = T // NW


def _gate_body(x_ref, wg_ref, bg_ref, ws_ref, bs_ref,
               pos_ref, meta_ref, f_ref, p_ref, wse_ref, bse_ref):
    x = x_ref[0]
    logits = jnp.dot(x, wg_ref[...], preferred_element_type=jnp.float32)
    s = jax.nn.sigmoid(logits + bg_ref[...])
    m = jnp.max(s, axis=1, keepdims=True)
    lane_e = lax.broadcasted_iota(jnp.int32, (1, E), 1)
    cand = jnp.where(s >= m, lane_e, E)
    e_t = jnp.min(cand, axis=1, keepdims=True)
    onehot = (lane_e == e_t).astype(jnp.float32)

    denom = jnp.sum(s, axis=1, keepdims=True)
    s_sel = m / denom
    counts = jnp.sum(onehot, axis=0, keepdims=True)
    f_ref[...] = jnp.float32(T) - counts
    sel_per_e = jnp.sum(onehot * s_sel, axis=0, keepdims=True)
    p_ref[...] = jnp.sum(s_sel) - sel_per_e

    ws = ws_ref[...]
    bs = bs_ref[...]
    wse_ref[...] = (ws[:, :C] + ws[:, C:]).astype(jnp.bfloat16)
    bse_ref[...] = bs[:, :C] + bs[:, C:]

    tiles = jnp.floor((counts + jnp.float32(TM - 1)) * jnp.float32(1.0 / TM))
    r16 = lax.broadcasted_iota(jnp.int32, (E, E), 0)
    c16 = lax.broadcasted_iota(jnp.int32, (E, E), 1)
    excl = (r16 < c16).astype(jnp.float32)
    ts_row = jnp.dot(tiles, excl, preferred_element_type=jnp.float32)
    nu = jnp.sum(tiles, axis=1, keepdims=True)

    BL = 256
    r_b = lax.broadcasted_iota(jnp.int32, (BL, BL), 0)
    c_b = lax.broadcasted_iota(jnp.int32, (BL, BL), 1)
    tri = (r_b >= c_b).astype(jnp.float32)
    ranks = []
    off = jnp.zeros((1, E), jnp.float32)
    for i in range(T // BL):
        blk = onehot[i * BL:(i + 1) * BL]
        cum = jnp.dot(tri, blk, preferred_element_type=jnp.float32) + off
        off = off + jnp.sum(blk, axis=0, keepdims=True)
        ranks.append(jnp.sum(blk * cum, axis=1, keepdims=True) - 1.0)
    rank = jnp.concatenate(ranks, axis=0)

    ts_t = jnp.sum(onehot * ts_row, axis=1, keepdims=True)
    pos_ref[...] = (jnp.float32(TM) * ts_t + rank).astype(jnp.int32)

    lane = lax.broadcasted_iota(jnp.int32, (1, 128), 1)
    nu_i = nu.astype(jnp.int32)
    g1 = jnp.minimum(lane, nu_i - 1)
    acc = jnp.zeros((1, 128), jnp.int32)
    for e in range(E):
        ts_e = ts_row[:, e:e + 1].astype(jnp.int32)
        acc = acc + (g1 >= ts_e).astype(jnp.int32)
    eot = acc - 1
    tidx = jnp.minimum(lane - 32, nu_i - 1)
    meta_ref[...] = jnp.where(lane < 32, eot,
                              jnp.where(lane < 64, tidx, nu_i))


def _run_gate(x3, Wg, bg2, Ws, bs2, interpret=False):
    return pl.pallas_call(
        _gate_body,
        out_shape=(
            jax.ShapeDtypeStruct((T, 1), jnp.int32),
            jax.ShapeDtypeStruct((1, 128), jnp.int32),
            jax.ShapeDtypeStruct((1, E), jnp.float32),
            jax.ShapeDtypeStruct((1, E), jnp.float32),
            jax.ShapeDtypeStruct((C, C), jnp.bfloat16),
            jax.ShapeDtypeStruct((1, C), jnp.float32),
        ),
        interpret=interpret,
    )(x3, Wg, bg2, Ws, bs2)


FH = FF // 2


def _ffn_body(meta_ref, x_ref, w1a_ref, w1b_ref, b1_ref,
              w2a_ref, w2b_ref, b2_ref, wse_ref, bse_ref, y_ref):
    g = pl.program_id(0)

    @pl.when(g < meta_ref[64])
    def _():
        e = meta_ref[g]
        xf = x_ref[...]
        xb = xf.astype(jnp.bfloat16)
        b1row = b1_ref[pl.ds(e, 1), :]
        ha = jnp.dot(xb, w1a_ref[0].astype(jnp.bfloat16),
                     preferred_element_type=jnp.float32)
        ha = jax.nn.gelu(ha + b1row[:, :FH]).astype(jnp.bfloat16)
        hb = jnp.dot(xb, w1b_ref[0].astype(jnp.bfloat16),
                     preferred_element_type=jnp.float32)
        hb = jax.nn.gelu(hb + b1row[:, FH:]).astype(jnp.bfloat16)
        y = (jnp.dot(ha, w2a_ref[0].astype(jnp.bfloat16),
                     preferred_element_type=jnp.float32)
             + jnp.dot(hb, w2b_ref[0].astype(jnp.bfloat16),
                       preferred_element_type=jnp.float32))
        shared = jnp.dot(xb, wse_ref[...], preferred_element_type=jnp.float32)
        y_ref[...] = xf + shared + y + b2_ref[pl.ds(e, 1), :] + bse_ref[...]


def _run_ffn(meta, xpad, W1, b1, W2, b2, wse, bse, interpret=False):
    grid_spec = pltpu.PrefetchScalarGridSpec(
        num_scalar_prefetch=1,
        grid=(NT,),
        in_specs=[
            pl.BlockSpec((TM, C), lambda g, m: (m[32 + g], 0)),
            pl.BlockSpec((1, C, FH), lambda g, m: (m[g], 0, 0)),
            pl.BlockSpec((1, C, FH), lambda g, m: (m[g], 0, 1)),
            pl.BlockSpec((E, FF), lambda g, m: (0, 0)),
            pl.BlockSpec((1, FH, C), lambda g, m: (m[g], 0, 0)),
            pl.BlockSpec((1, FH, C), lambda g, m: (m[g], 1, 0)),
            pl.BlockSpec((E, C), lambda g, m: (0, 0)),
            pl.BlockSpec((C, C), lambda g, m: (0, 0)),
            pl.BlockSpec((1, C), lambda g, m: (0, 0)),
        ],
        out_specs=pl.BlockSpec((TM, C), lambda g, m: (m[32 + g], 0)),
    )
    return pl.pallas_call(
        _ffn_body,
        grid_spec=grid_spec,
        out_shape=jax.ShapeDtypeStruct((NPAD, C), jnp.float32),
        compiler_params=pltpu.CompilerParams(
            dimension_semantics=("arbitrary",)),
        interpret=interpret,
    )(meta, xpad, W1, W1, b1, W2, W2, b2, wse, bse)


@functools.cache
def _sc_kernels():
    mesh = plsc.VectorSubcoreMesh(core_axis_name="c", subcore_axis_name="s")

    @functools.partial(
        pl.kernel,
        out_type=jax.ShapeDtypeStruct((NPAD, C), jnp.float32),
        mesh=mesh,
        scratch_types=[pltpu.VMEM((TPW,), jnp.int32),
                       pltpu.VMEM((TPW, C), jnp.float32),
                       pltpu.SemaphoreType.DMA,
                       pltpu.SemaphoreType.DMA],
    )
    def _sc_scatter(x_hbm, pos_hbm, xpad_hbm, idx_v, buf_v, sem1, sem2):
        wid = lax.axis_index("s") * 2 + lax.axis_index("c")
        start = wid * TPW
        cpx = pltpu.async_copy(x_hbm.at[0, pl.ds(start, TPW)], buf_v, sem1)
        pltpu.sync_copy(pos_hbm.at[pl.ds(start, TPW)], idx_v)
        cpx.wait()
        pltpu.async_copy(buf_v, xpad_hbm.at[idx_v], sem2).wait()

    @functools.partial(
        pl.kernel,
        out_type=jax.ShapeDtypeStruct((1, T, C), jnp.float32),
        mesh=mesh,
        scratch_types=[pltpu.VMEM((TPW,), jnp.int32),
                       pltpu.VMEM((TPW, C), jnp.float32),
                       pltpu.SemaphoreType.DMA],
    )
    def _sc_gather(ypad_hbm, pos_hbm, res_hbm, idx_v, buf_v, sem):
        wid = lax.axis_index("s") * 2 + lax.axis_index("c")
        start = wid * TPW
        pltpu.sync_copy(pos_hbm.at[pl.ds(start, TPW)], idx_v)
        pltpu.async_copy(ypad_hbm.at[idx_v], buf_v, sem).wait()
        pltpu.sync_copy(buf_v, res_hbm.at[0, pl.ds(start, TPW)])

    return _sc_scatter, _sc_gather


def kernel(x, Ws, bs, Wg, bg, W1, b1, W2, b2):
    pos, meta, f, p, wse, bse = _run_gate(
        x, Wg, bg.reshape(1, -1), Ws, bs.reshape(1, -1))
    pos1 = pos.reshape(T)
    _sc_scatter, _sc_gather = _sc_kernels()
    xpad = _sc_scatter(x, pos1)
    ypad = _run_ffn(meta.reshape(128), xpad, W1, b1, W2, b2, wse, bse)
    res = _sc_gather(ypad, pos1)
    return res, (f, p)
